# baseline (speedup 1.0000x reference)
import functools
import math

import jax
import jax.numpy as jnp
from jax import lax
from jax.experimental import pallas as pl
from jax.experimental.pallas import tpu as pltpu
from jax.experimental.pallas import tpu_sc as plsc

N = 10000
E = 320000
D = 128
H = 4
HD = H * D

NC = 2
NS = 16
NW = NC * NS
EPW = E // NW
CA = 80
CM = 40
RPT = N // NS
ZR = 25
SCALE = 1.0 / math.sqrt(float(D))

_MESH = plsc.VectorSubcoreMesh(core_axis_name="c", subcore_axis_name="s",
                               num_cores=NC, num_subcores=NS)


def _wid():
    return lax.axis_index("s") * NC + lax.axis_index("c")


def _zero_acc(zbuf, acc, width):
    sid = lax.axis_index("s")

    def zrow(i, _):
        for c in range(width // 16):
            zbuf[i, pl.ds(c * 16, 16)] = jnp.zeros((16,), jnp.float32)
        return 0

    lax.fori_loop(0, ZR, zrow, 0)
    for p in range(RPT // ZR):
        pltpu.sync_copy(zbuf, acc.at[pl.ds(sid * RPT + p * ZR, ZR)])


def _export_acc(acc, out_hbm):
    cid = lax.axis_index("c")
    sid = lax.axis_index("s")
    pltpu.sync_copy(acc.at[pl.ds(sid * RPT, RPT)],
                    out_hbm.at[cid, pl.ds(sid * RPT, RPT)])


@functools.partial(
    pl.kernel,
    out_type=jax.ShapeDtypeStruct((NC, N, 16), jnp.float32),
    mesh=_MESH,
    compiler_params=pltpu.CompilerParams(use_tc_tiling_on_sc=False, needs_layout_passes=False),
    scratch_types=[
        pltpu.VMEM((CA,), jnp.int32),
        pltpu.VMEM((CA, 16), jnp.float32),
        pltpu.VMEM((ZR, 16), jnp.float32),
        pltpu.VMEM_SHARED((N, 16), jnp.float32),
    ],
)
def _sc_deg(dst_hbm, out_hbm, didx, ones, zbuf, acc):
    wid = _wid()

    def orow(i, _):
        ones[i, :] = jnp.ones((16,), jnp.float32)
        return 0

    lax.fori_loop(0, CA, orow, 0)
    _zero_acc(zbuf, acc, 16)
    plsc.subcore_barrier()

    def chunk(j, _):
        pltpu.sync_copy(dst_hbm.at[wid, pl.ds(j * CA, CA)], didx)
        pltpu.sync_copy(ones, acc.at[didx], add=True)
        return 0

    lax.fori_loop(0, EPW // CA, chunk, 0)
    plsc.subcore_barrier()
    _export_acc(acc, out_hbm)


@functools.partial(
    pl.kernel,
    out_type=jax.ShapeDtypeStruct((NC, N, D), jnp.float32),
    mesh=_MESH,
    compiler_params=pltpu.CompilerParams(use_tc_tiling_on_sc=False, needs_layout_passes=False),
    scratch_types=[
        pltpu.VMEM((CA,), jnp.int32),
        pltpu.VMEM((CA,), jnp.int32),
        pltpu.VMEM((CA, D), jnp.float32),
        pltpu.VMEM((ZR, D), jnp.float32),
        pltpu.VMEM_SHARED((N, D), jnp.float32),
    ],
)
def _sc_gcn(xs_hbm, src_hbm, dst_hbm, out_hbm, sidx, didx, rows, zbuf, acc):
    wid = _wid()
    _zero_acc(zbuf, acc, D)
    plsc.subcore_barrier()

    def chunk(j, _):
        pltpu.sync_copy(src_hbm.at[wid, pl.ds(j * CA, CA)], sidx)
        pltpu.sync_copy(dst_hbm.at[wid, pl.ds(j * CA, CA)], didx)
        pltpu.sync_copy(xs_hbm.at[sidx], rows)
        pltpu.sync_copy(rows, acc.at[didx], add=True)
        return 0

    lax.fori_loop(0, EPW // CA, chunk, 0)
    plsc.subcore_barrier()
    _export_acc(acc, out_hbm)


CB = 40


@functools.partial(
    pl.kernel,
    out_type=[
        jax.ShapeDtypeStruct((NW, EPW, 16), jnp.float32),
        jax.ShapeDtypeStruct((NC, N, 16), jnp.float32),
    ],
    mesh=_MESH,
    compiler_params=pltpu.CompilerParams(use_tc_tiling_on_sc=False, needs_layout_passes=False),
    scratch_types=[
        pltpu.VMEM((2, CB), jnp.int32),
        pltpu.VMEM((2, CB), jnp.int32),
        pltpu.VMEM((2, CB, HD), jnp.float32),
        pltpu.VMEM((2, CB, HD), jnp.float32),
        pltpu.VMEM((CB, 16), jnp.float32),
        pltpu.VMEM((ZR, 16), jnp.float32),
        pltpu.VMEM_SHARED((N, 16), jnp.float32),
        pltpu.SemaphoreType.DMA,
        pltpu.SemaphoreType.DMA,
    ],
)
def _sc_attn(q_hbm, k_hbm, src_hbm, dst_hbm, ex_hbm, den_hbm,
             sidx, didx, qb, kb, exb, zbuf, acc, qsem, ksem):
    wid = _wid()
    _zero_acc(zbuf, acc, 16)
    plsc.subcore_barrier()
    lane = lax.iota(jnp.int32, 16)
    nch = EPW // CB

    def fetch(j, p):
        pltpu.sync_copy(src_hbm.at[wid, pl.ds(j * CB, CB)], sidx.at[p])
        pltpu.sync_copy(dst_hbm.at[wid, pl.ds(j * CB, CB)], didx.at[p])
        pltpu.async_copy(q_hbm.at[didx.at[p]], qb.at[p], qsem)
        pltpu.async_copy(k_hbm.at[sidx.at[p]], kb.at[p], ksem)

    fetch(0, 0)

    def chunk(j, _):
        p = lax.rem(j, 2)

        @pl.when(j + 1 < nch)
        def _():
            fetch(j + 1, 1 - p)

        pltpu.make_async_copy(q_hbm.at[didx.at[p]], qb.at[p], qsem).wait()
        pltpu.make_async_copy(k_hbm.at[sidx.at[p]], kb.at[p], ksem).wait()

        def edge(e, _):
            val = jnp.zeros((16,), jnp.float32)
            for h in range(H):
                a = jnp.zeros((16,), jnp.float32)
                for c in range(D // 16):
                    off = h * D + c * 16
                    a = a + qb[p, e, pl.ds(off, 16)] * kb[p, e, pl.ds(off, 16)]
                val = jnp.where(lane == h, jnp.sum(a) * SCALE, val)
            exb[e, :] = jnp.exp(val)
            return 0

        for e_static in range(CB):
            edge(e_static, 0)
        pltpu.sync_copy(exb, ex_hbm.at[wid, pl.ds(j * CB, CB)])
        pltpu.sync_copy(exb, acc.at[didx.at[p]], add=True)
        return 0

    lax.fori_loop(0, nch, chunk, 0)
    plsc.subcore_barrier()
    _export_acc(acc, den_hbm)


@functools.partial(
    pl.kernel,
    out_type=jax.ShapeDtypeStruct((NC, N, D), jnp.float32),
    mesh=_MESH,
    compiler_params=pltpu.CompilerParams(use_tc_tiling_on_sc=False, needs_layout_passes=False),
    scratch_types=[
        pltpu.VMEM((CM,), jnp.int32),
        pltpu.VMEM((CM,), jnp.int32),
        pltpu.VMEM((CM, HD), jnp.float32),
        pltpu.VMEM((CM, 16), jnp.float32),
        pltpu.VMEM((CM, 16), jnp.float32),
        pltpu.VMEM((CM, D), jnp.float32),
        pltpu.VMEM((ZR, D), jnp.float32),
        pltpu.VMEM_SHARED((N, D), jnp.float32),
    ],
)
def _sc_agg(v_hbm, ex_hbm, den_hbm, src_hbm, dst_hbm, out_hbm,
            sidx, didx, vb, exb, db, mb, zbuf, acc):
    wid = _wid()
    _zero_acc(zbuf, acc, D)
    plsc.subcore_barrier()

    def chunk(j, _):
        pltpu.sync_copy(src_hbm.at[wid, pl.ds(j * CM, CM)], sidx)
        pltpu.sync_copy(dst_hbm.at[wid, pl.ds(j * CM, CM)], didx)
        pltpu.sync_copy(ex_hbm.at[wid, pl.ds(j * CM, CM)], exb)
        pltpu.sync_copy(v_hbm.at[sidx], vb)
        pltpu.sync_copy(den_hbm.at[didx], db)

        def edge(e, _):
            w = exb[e, :] * db[e, :]
            ws = [jnp.full((16,), w[h]) for h in range(H)]
            for c in range(D // 16):
                a = jnp.zeros((16,), jnp.float32)
                for h in range(H):
                    a = a + ws[h] * vb[e, pl.ds(h * D + c * 16, 16)]
                mb[e, pl.ds(c * 16, 16)] = a
            return 0

        for e_static in range(CM):
            edge(e_static, 0)
        pltpu.sync_copy(mb, acc.at[didx], add=True)
        return 0

    lax.fori_loop(0, EPW // CM, chunk, 0)
    plsc.subcore_barrier()
    _export_acc(acc, out_hbm)


_BN = 1000
_GRID = N // _BN


def _tc_prep_body(f_ref, wg_ref, degp_ref, xs_ref, dinv_ref):
    x = jnp.dot(f_ref[...], wg_ref[...], preferred_element_type=jnp.float32)
    deg = degp_ref[0, :, 0] + degp_ref[1, :, 0] + 1.0
    dinv = lax.rsqrt(deg)
    xs_ref[...] = x * dinv[:, None]
    dinv_ref[...] = jnp.broadcast_to(dinv[:, None], (_BN, 16))


def _tc_prep(f_all, w_gcn, degp):
    return pl.pallas_call(
        _tc_prep_body,
        grid=(_GRID,),
        in_specs=[
            pl.BlockSpec((_BN, D), lambda i: (i, 0)),
            pl.BlockSpec((D, D), lambda i: (0, 0)),
            pl.BlockSpec((NC, _BN, 16), lambda i: (0, i, 0)),
        ],
        out_specs=[
            pl.BlockSpec((_BN, D), lambda i: (i, 0)),
            pl.BlockSpec((_BN, 16), lambda i: (i, 0)),
        ],
        out_shape=[
            jax.ShapeDtypeStruct((N, D), jnp.float32),
            jax.ShapeDtypeStruct((N, 16), jnp.float32),
        ],
    )(f_all, w_gcn, degp)


def _tc_mid_body(s_ref, xs_ref, dinv_ref, bg_ref, wq_ref, bq_ref, wk_ref,
                 bk_ref, wv_ref, bv_ref, ws_ref, bs_ref,
                 q_ref, k_ref, v_ref, xr_ref):
    dinv = dinv_ref[...][:, 0:1]
    gcn = dinv * (s_ref[0] + s_ref[1] + xs_ref[...]) + bg_ref[...]
    gcn = jnp.maximum(gcn, 0.0)
    q_ref[...] = jnp.dot(gcn, wq_ref[...], preferred_element_type=jnp.float32) + bq_ref[...]
    k_ref[...] = jnp.dot(gcn, wk_ref[...], preferred_element_type=jnp.float32) + bk_ref[...]
    v_ref[...] = jnp.dot(gcn, wv_ref[...], preferred_element_type=jnp.float32) + bv_ref[...]
    xr_ref[...] = jnp.dot(gcn, ws_ref[...], preferred_element_type=jnp.float32) + bs_ref[...]


def _tc_mid(s, xs, dinv16, b_gcn, wq, bq, wk, bk, wv, bv, w_skip, b_skip):
    full = lambda r, c: pl.BlockSpec((r, c), lambda i: (0, 0))
    rows = lambda c: pl.BlockSpec((_BN, c), lambda i: (i, 0))
    return pl.pallas_call(
        _tc_mid_body,
        grid=(_GRID,),
        in_specs=[
            pl.BlockSpec((NC, _BN, D), lambda i: (0, i, 0)),
            rows(D), rows(16), full(1, D),
            full(D, HD), full(1, HD), full(D, HD), full(1, HD),
            full(D, HD), full(1, HD), full(D, D), full(1, D),
        ],
        out_specs=[rows(HD), rows(HD), rows(HD), rows(D)],
        out_shape=[
            jax.ShapeDtypeStruct((N, HD), jnp.float32),
            jax.ShapeDtypeStruct((N, HD), jnp.float32),
            jax.ShapeDtypeStruct((N, HD), jnp.float32),
            jax.ShapeDtypeStruct((N, D), jnp.float32),
        ],
    )(s, xs, dinv16, b_gcn, wq, bq, wk, bk, wv, bv, w_skip, b_skip)


def _tc_den_body(dp_ref, den_ref):
    den_ref[...] = (1.0 / H) / (dp_ref[0] + dp_ref[1] + 1e-16)


def _tc_den(denp):
    return pl.pallas_call(
        _tc_den_body,
        grid=(_GRID,),
        in_specs=[pl.BlockSpec((NC, _BN, 16), lambda i: (0, i, 0))],
        out_specs=pl.BlockSpec((_BN, 16), lambda i: (i, 0)),
        out_shape=jax.ShapeDtypeStruct((N, 16), jnp.float32),
    )(denp)


def _tc_post_body(op_ref, xr_ref, wb_ref, wc_ref, bc_ref, conv_ref):
    out = op_ref[0] + op_ref[1]
    xr = xr_ref[...]
    wb = wb_ref[...]
    logit = (jnp.sum(out * wb[0:1, :], axis=-1, keepdims=True)
             + jnp.sum(xr * wb[1:2, :], axis=-1, keepdims=True)
             + jnp.sum((out - xr) * wb[2:3, :], axis=-1, keepdims=True))
    beta = jax.nn.sigmoid(logit)
    o = beta * xr + (1.0 - beta) * out
    tf = jnp.maximum(o, 0.0)
    conv = lax.dot_general(wc_ref[...], tf, (((1,), (1,)), ((), ())),
                           preferred_element_type=jnp.float32)
    conv_ref[...] = conv + bc_ref[...][:, 0:1]


def _tc_post(op, xr, wb3, w_cnn, b_cnn2):
    npad = 10240
    opp = jnp.pad(op, ((0, 0), (0, npad - N), (0, 0)))
    xrp = jnp.pad(xr, ((0, npad - N), (0, 0)))
    bp = 1024
    conv = pl.pallas_call(
        _tc_post_body,
        grid=(npad // bp,),
        in_specs=[
            pl.BlockSpec((NC, bp, D), lambda i: (0, i, 0)),
            pl.BlockSpec((bp, D), lambda i: (i, 0)),
            pl.BlockSpec((3, D), lambda i: (0, 0)),
            pl.BlockSpec((D, D), lambda i: (0, 0)),
            pl.BlockSpec((D, 16), lambda i: (0, 0)),
        ],
        out_specs=pl.BlockSpec((D, bp), lambda i: (0, i)),
        out_shape=jax.ShapeDtypeStruct((D, npad), jnp.float32),
    )(opp, xrp, wb3, w_cnn, b_cnn2)
    return conv[:, :N]


def kernel(f_all, edge_index, W_gcn, b_gcn, Wq, bq, Wk, bk, Wv, bv,
           W_skip, b_skip, W_beta, W_cnn, b_cnn):
    src = edge_index[0].reshape(NW, EPW)
    dst = edge_index[1].reshape(NW, EPW)

    degp = _sc_deg(dst)
    xs, dinv16 = _tc_prep(f_all, W_gcn, degp)
    s = _sc_gcn(xs, src, dst)
    q, k, v, xr = _tc_mid(s, xs, dinv16, b_gcn.reshape(1, D),
                          Wq, bq.reshape(1, HD), Wk, bk.reshape(1, HD),
                          Wv, bv.reshape(1, HD), W_skip, b_skip.reshape(1, D))
    ex, denp = _sc_attn(q, k, src, dst)
    den = _tc_den(denp)
    op = _sc_agg(v, ex, den, src, dst)
    wb3 = W_beta.reshape(3, D)
    b_cnn2 = jnp.broadcast_to(b_cnn[:, None], (D, 16))
    conv = _tc_post(op, xr, wb3, W_cnn, b_cnn2)
    return conv.reshape(1, N, D)


# speedup vs baseline: 1.3163x; 1.3163x over previous
import functools
import math

import jax
import jax.numpy as jnp
from jax import lax
from jax.experimental import pallas as pl
from jax.experimental.pallas import tpu as pltpu
from jax.experimental.pallas import tpu_sc as plsc

N = 10000
E = 320000
D = 128
H = 4
HD = H * D

NC = 2
NS = 16
NW = NC * NS
EPW = E // NW
CA = 80
CM = 40
RPT = N // NS
ZR = 25
SCALE = 1.0 / math.sqrt(float(D))

_MESH = plsc.VectorSubcoreMesh(core_axis_name="c", subcore_axis_name="s",
                               num_cores=NC, num_subcores=NS)


def _wid():
    return lax.axis_index("s") * NC + lax.axis_index("c")


def _zero_acc(zbuf, acc, width):
    sid = lax.axis_index("s")

    def zrow(i, _):
        for c in range(width // 16):
            zbuf[i, pl.ds(c * 16, 16)] = jnp.zeros((16,), jnp.float32)
        return 0

    lax.fori_loop(0, ZR, zrow, 0)
    for p in range(RPT // ZR):
        pltpu.sync_copy(zbuf, acc.at[pl.ds(sid * RPT + p * ZR, ZR)])


def _export_acc(acc, out_hbm):
    cid = lax.axis_index("c")
    sid = lax.axis_index("s")
    pltpu.sync_copy(acc.at[pl.ds(sid * RPT, RPT)],
                    out_hbm.at[cid, pl.ds(sid * RPT, RPT)])


@functools.partial(
    pl.kernel,
    out_type=jax.ShapeDtypeStruct((NC, N, 16), jnp.float32),
    mesh=_MESH,
    compiler_params=pltpu.CompilerParams(use_tc_tiling_on_sc=False, needs_layout_passes=False),
    scratch_types=[
        pltpu.VMEM((CA,), jnp.int32),
        pltpu.VMEM((CA, 16), jnp.float32),
        pltpu.VMEM((ZR, 16), jnp.float32),
        pltpu.VMEM_SHARED((N, 16), jnp.float32),
    ],
)
def _sc_deg(dst_hbm, out_hbm, didx, ones, zbuf, acc):
    wid = _wid()

    def orow(i, _):
        ones[i, :] = jnp.ones((16,), jnp.float32)
        return 0

    lax.fori_loop(0, CA, orow, 0)
    _zero_acc(zbuf, acc, 16)
    plsc.subcore_barrier()

    def chunk(j, _):
        pltpu.sync_copy(dst_hbm.at[wid, pl.ds(j * CA, CA)], didx)
        pltpu.sync_copy(ones, acc.at[didx], add=True)
        return 0

    lax.fori_loop(0, EPW // CA, chunk, 0)
    plsc.subcore_barrier()
    _export_acc(acc, out_hbm)


@functools.partial(
    pl.kernel,
    out_type=jax.ShapeDtypeStruct((NC, N, D), jnp.float32),
    mesh=_MESH,
    compiler_params=pltpu.CompilerParams(use_tc_tiling_on_sc=False, needs_layout_passes=False),
    scratch_types=[
        pltpu.VMEM((2, CA), jnp.int32),
        pltpu.VMEM((2, CA), jnp.int32),
        pltpu.VMEM((2, CA, D), jnp.float32),
        pltpu.VMEM((ZR, D), jnp.float32),
        pltpu.VMEM_SHARED((N, D), jnp.float32),
        pltpu.SemaphoreType.DMA,
    ],
)
def _sc_gcn(xs_hbm, src_hbm, dst_hbm, out_hbm, sidx, didx, rows, zbuf, acc,
            gsem):
    wid = _wid()
    _zero_acc(zbuf, acc, D)
    plsc.subcore_barrier()
    nch = EPW // CA

    def fetch(j, p):
        pltpu.sync_copy(src_hbm.at[wid, pl.ds(j * CA, CA)], sidx.at[p])
        pltpu.sync_copy(dst_hbm.at[wid, pl.ds(j * CA, CA)], didx.at[p])
        pltpu.async_copy(xs_hbm.at[sidx.at[p]], rows.at[p], gsem)

    fetch(0, 0)

    def chunk(j, _):
        p = lax.rem(j, 2)

        @pl.when(j + 1 < nch)
        def _():
            fetch(j + 1, 1 - p)

        pltpu.make_async_copy(xs_hbm.at[sidx.at[p]], rows.at[p], gsem).wait()
        pltpu.sync_copy(rows.at[p], acc.at[didx.at[p]], add=True)
        return 0

    lax.fori_loop(0, nch, chunk, 0)
    plsc.subcore_barrier()
    _export_acc(acc, out_hbm)


CB = 40


@functools.partial(
    pl.kernel,
    out_type=[
        jax.ShapeDtypeStruct((NW, EPW, 16), jnp.float32),
        jax.ShapeDtypeStruct((NC, N, 16), jnp.float32),
    ],
    mesh=_MESH,
    compiler_params=pltpu.CompilerParams(use_tc_tiling_on_sc=False, needs_layout_passes=False),
    scratch_types=[
        pltpu.VMEM((2, CB), jnp.int32),
        pltpu.VMEM((2, CB), jnp.int32),
        pltpu.VMEM((2, CB, HD), jnp.float32),
        pltpu.VMEM((2, CB, HD), jnp.float32),
        pltpu.VMEM((CB, 16), jnp.float32),
        pltpu.VMEM((ZR, 16), jnp.float32),
        pltpu.VMEM_SHARED((N, 16), jnp.float32),
        pltpu.SemaphoreType.DMA,
        pltpu.SemaphoreType.DMA,
    ],
)
def _sc_attn(q_hbm, k_hbm, src_hbm, dst_hbm, ex_hbm, den_hbm,
             sidx, didx, qb, kb, exb, zbuf, acc, qsem, ksem):
    wid = _wid()
    _zero_acc(zbuf, acc, 16)
    plsc.subcore_barrier()
    lane = lax.iota(jnp.int32, 16)
    nch = EPW // CB

    def fetch(j, p):
        pltpu.sync_copy(src_hbm.at[wid, pl.ds(j * CB, CB)], sidx.at[p])
        pltpu.sync_copy(dst_hbm.at[wid, pl.ds(j * CB, CB)], didx.at[p])
        pltpu.async_copy(q_hbm.at[didx.at[p]], qb.at[p], qsem)
        pltpu.async_copy(k_hbm.at[sidx.at[p]], kb.at[p], ksem)

    fetch(0, 0)

    def chunk(j, _):
        p = lax.rem(j, 2)

        @pl.when(j + 1 < nch)
        def _():
            fetch(j + 1, 1 - p)

        pltpu.make_async_copy(q_hbm.at[didx.at[p]], qb.at[p], qsem).wait()
        pltpu.make_async_copy(k_hbm.at[sidx.at[p]], kb.at[p], ksem).wait()

        def edge(e, _):
            val = jnp.zeros((16,), jnp.float32)
            for h in range(H):
                a = jnp.zeros((16,), jnp.float32)
                for c in range(D // 16):
                    off = h * D + c * 16
                    a = a + qb[p, e, pl.ds(off, 16)] * kb[p, e, pl.ds(off, 16)]
                val = jnp.where(lane == h, jnp.sum(a) * SCALE, val)
            exb[e, :] = jnp.exp(val)
            return 0

        lax.fori_loop(0, CB, edge, 0, unroll=4)
        pltpu.sync_copy(exb, ex_hbm.at[wid, pl.ds(j * CB, CB)])
        pltpu.sync_copy(exb, acc.at[didx.at[p]], add=True)
        return 0

    lax.fori_loop(0, nch, chunk, 0)
    plsc.subcore_barrier()
    _export_acc(acc, den_hbm)


@functools.partial(
    pl.kernel,
    out_type=jax.ShapeDtypeStruct((NC, N, D), jnp.float32),
    mesh=_MESH,
    compiler_params=pltpu.CompilerParams(use_tc_tiling_on_sc=False, needs_layout_passes=False),
    scratch_types=[
        pltpu.VMEM((CM,), jnp.int32),
        pltpu.VMEM((CM,), jnp.int32),
        pltpu.VMEM((CM, HD), jnp.float32),
        pltpu.VMEM((CM, 16), jnp.float32),
        pltpu.VMEM((CM, 16), jnp.float32),
        pltpu.VMEM((CM, D), jnp.float32),
        pltpu.VMEM((ZR, D), jnp.float32),
        pltpu.VMEM_SHARED((N, D), jnp.float32),
    ],
)
def _sc_agg(v_hbm, ex_hbm, den_hbm, src_hbm, dst_hbm, out_hbm,
            sidx, didx, vb, exb, db, mb, zbuf, acc):
    wid = _wid()
    _zero_acc(zbuf, acc, D)
    plsc.subcore_barrier()

    def chunk(j, _):
        pltpu.sync_copy(src_hbm.at[wid, pl.ds(j * CM, CM)], sidx)
        pltpu.sync_copy(dst_hbm.at[wid, pl.ds(j * CM, CM)], didx)
        pltpu.sync_copy(ex_hbm.at[wid, pl.ds(j * CM, CM)], exb)
        pltpu.sync_copy(v_hbm.at[sidx], vb)
        pltpu.sync_copy(den_hbm.at[didx], db)

        def edge(e, _):
            w = exb[e, :] * db[e, :]
            ws = [jnp.full((16,), w[h]) for h in range(H)]
            for c in range(D // 16):
                a = jnp.zeros((16,), jnp.float32)
                for h in range(H):
                    a = a + ws[h] * vb[e, pl.ds(h * D + c * 16, 16)]
                mb[e, pl.ds(c * 16, 16)] = a
            return 0

        for e_static in range(CM):
            edge(e_static, 0)
        pltpu.sync_copy(mb, acc.at[didx], add=True)
        return 0

    lax.fori_loop(0, EPW // CM, chunk, 0)
    plsc.subcore_barrier()
    _export_acc(acc, out_hbm)


_BN = 1000
_GRID = N // _BN


def _tc_prep_body(f_ref, wg_ref, degp_ref, xs_ref, dinv_ref):
    x = jnp.dot(f_ref[...], wg_ref[...], preferred_element_type=jnp.float32)
    deg = degp_ref[0, :, 0] + degp_ref[1, :, 0] + 1.0
    dinv = lax.rsqrt(deg)
    xs_ref[...] = x * dinv[:, None]
    dinv_ref[...] = jnp.broadcast_to(dinv[:, None], (_BN, 16))


def _tc_prep(f_all, w_gcn, degp):
    return pl.pallas_call(
        _tc_prep_body,
        grid=(_GRID,),
        in_specs=[
            pl.BlockSpec((_BN, D), lambda i: (i, 0)),
            pl.BlockSpec((D, D), lambda i: (0, 0)),
            pl.BlockSpec((NC, _BN, 16), lambda i: (0, i, 0)),
        ],
        out_specs=[
            pl.BlockSpec((_BN, D), lambda i: (i, 0)),
            pl.BlockSpec((_BN, 16), lambda i: (i, 0)),
        ],
        out_shape=[
            jax.ShapeDtypeStruct((N, D), jnp.float32),
            jax.ShapeDtypeStruct((N, 16), jnp.float32),
        ],
    )(f_all, w_gcn, degp)


def _tc_mid_body(s_ref, xs_ref, dinv_ref, bg_ref, wq_ref, bq_ref, wk_ref,
                 bk_ref, wv_ref, bv_ref, ws_ref, bs_ref,
                 q_ref, k_ref, v_ref, xr_ref):
    dinv = dinv_ref[...][:, 0:1]
    gcn = dinv * (s_ref[0] + s_ref[1] + xs_ref[...]) + bg_ref[...]
    gcn = jnp.maximum(gcn, 0.0)
    q_ref[...] = jnp.dot(gcn, wq_ref[...], preferred_element_type=jnp.float32) + bq_ref[...]
    k_ref[...] = jnp.dot(gcn, wk_ref[...], preferred_element_type=jnp.float32) + bk_ref[...]
    v_ref[...] = jnp.dot(gcn, wv_ref[...], preferred_element_type=jnp.float32) + bv_ref[...]
    xr_ref[...] = jnp.dot(gcn, ws_ref[...], preferred_element_type=jnp.float32) + bs_ref[...]


def _tc_mid(s, xs, dinv16, b_gcn, wq, bq, wk, bk, wv, bv, w_skip, b_skip):
    full = lambda r, c: pl.BlockSpec((r, c), lambda i: (0, 0))
    rows = lambda c: pl.BlockSpec((_BN, c), lambda i: (i, 0))
    return pl.pallas_call(
        _tc_mid_body,
        grid=(_GRID,),
        in_specs=[
            pl.BlockSpec((NC, _BN, D), lambda i: (0, i, 0)),
            rows(D), rows(16), full(1, D),
            full(D, HD), full(1, HD), full(D, HD), full(1, HD),
            full(D, HD), full(1, HD), full(D, D), full(1, D),
        ],
        out_specs=[rows(HD), rows(HD), rows(HD), rows(D)],
        out_shape=[
            jax.ShapeDtypeStruct((N, HD), jnp.float32),
            jax.ShapeDtypeStruct((N, HD), jnp.float32),
            jax.ShapeDtypeStruct((N, HD), jnp.float32),
            jax.ShapeDtypeStruct((N, D), jnp.float32),
        ],
    )(s, xs, dinv16, b_gcn, wq, bq, wk, bk, wv, bv, w_skip, b_skip)


def _tc_den_body(dp_ref, den_ref):
    den_ref[...] = (1.0 / H) / (dp_ref[0] + dp_ref[1] + 1e-16)


def _tc_den(denp):
    return pl.pallas_call(
        _tc_den_body,
        grid=(_GRID,),
        in_specs=[pl.BlockSpec((NC, _BN, 16), lambda i: (0, i, 0))],
        out_specs=pl.BlockSpec((_BN, 16), lambda i: (i, 0)),
        out_shape=jax.ShapeDtypeStruct((N, 16), jnp.float32),
    )(denp)


def _tc_post_body(op_ref, xr_ref, wb_ref, wc_ref, bc_ref, conv_ref):
    out = op_ref[0] + op_ref[1]
    xr = xr_ref[...]
    wb = wb_ref[...]
    logit = (jnp.sum(out * wb[0:1, :], axis=-1, keepdims=True)
             + jnp.sum(xr * wb[1:2, :], axis=-1, keepdims=True)
             + jnp.sum((out - xr) * wb[2:3, :], axis=-1, keepdims=True))
    beta = jax.nn.sigmoid(logit)
    o = beta * xr + (1.0 - beta) * out
    tf = jnp.maximum(o, 0.0)
    conv = lax.dot_general(wc_ref[...], tf, (((1,), (1,)), ((), ())),
                           preferred_element_type=jnp.float32)
    conv_ref[...] = conv + bc_ref[...][:, 0:1]


def _tc_post(op, xr, wb3, w_cnn, b_cnn2):
    npad = 10240
    opp = jnp.pad(op, ((0, 0), (0, npad - N), (0, 0)))
    xrp = jnp.pad(xr, ((0, npad - N), (0, 0)))
    bp = 1024
    conv = pl.pallas_call(
        _tc_post_body,
        grid=(npad // bp,),
        in_specs=[
            pl.BlockSpec((NC, bp, D), lambda i: (0, i, 0)),
            pl.BlockSpec((bp, D), lambda i: (i, 0)),
            pl.BlockSpec((3, D), lambda i: (0, 0)),
            pl.BlockSpec((D, D), lambda i: (0, 0)),
            pl.BlockSpec((D, 16), lambda i: (0, 0)),
        ],
        out_specs=pl.BlockSpec((D, bp), lambda i: (0, i)),
        out_shape=jax.ShapeDtypeStruct((D, npad), jnp.float32),
    )(opp, xrp, wb3, w_cnn, b_cnn2)
    return conv[:, :N]


def kernel(f_all, edge_index, W_gcn, b_gcn, Wq, bq, Wk, bk, Wv, bv,
           W_skip, b_skip, W_beta, W_cnn, b_cnn):
    src = edge_index[0].reshape(NW, EPW)
    dst = edge_index[1].reshape(NW, EPW)

    degp = _sc_deg(dst)
    xs, dinv16 = _tc_prep(f_all, W_gcn, degp)
    s = _sc_gcn(xs, src, dst)
    q, k, v, xr = _tc_mid(s, xs, dinv16, b_gcn.reshape(1, D),
                          Wq, bq.reshape(1, HD), Wk, bk.reshape(1, HD),
                          Wv, bv.reshape(1, HD), W_skip, b_skip.reshape(1, D))
    ex, denp = _sc_attn(q, k, src, dst)
    den = _tc_den(denp)
    op = _sc_agg(v, ex, den, src, dst)
    wb3 = W_beta.reshape(3, D)
    b_cnn2 = jnp.broadcast_to(b_cnn[:, None], (D, 16))
    conv = _tc_post(op, xr, wb3, W_cnn, b_cnn2)
    return conv.reshape(1, N, D)
